# baseline (speedup 1.0000x reference)
import functools

import jax
import jax.numpy as jnp
from jax import lax
from jax.experimental import pallas as pl
from jax.experimental.pallas import tpu as pltpu
from jax.experimental.pallas import tpu_sc as plsc

_BATCH = 16384
_D = 32
_L = 16


def _mf_body(uidx_hbm, iidx_hbm, *refs, n_per_w, num_cores):
    ucols = refs[:_D]
    icols = refs[_D:2 * _D]
    (ub_flat, ib_flat, out_hbm,
     uidx_v, iidx_v, ue_w, ie_w, ub_w, ib_w, out_v,
     sem_ue, sem_ie, sem_ub, sem_ib) = refs[2 * _D:]
    wid = lax.axis_index("s") * num_cores + lax.axis_index("c")
    base = wid * n_per_w

    pltpu.sync_copy(uidx_hbm.at[pl.ds(base, n_per_w)], uidx_v)
    pltpu.sync_copy(iidx_hbm.at[pl.ds(base, n_per_w)], iidx_v)

    descs = [
        pltpu.async_copy(ub_flat.at[uidx_v], ub_w, sem_ub),
        pltpu.async_copy(ib_flat.at[iidx_v], ib_w, sem_ib),
    ]
    for d in range(_D):
        descs.append(pltpu.async_copy(
            ucols[d].at[uidx_v], ue_w.at[pl.ds(d * n_per_w, n_per_w)],
            sem_ue))
        descs.append(pltpu.async_copy(
            icols[d].at[iidx_v], ie_w.at[pl.ds(d * n_per_w, n_per_w)],
            sem_ie))
    for c in descs:
        c.wait()

    def group(g, carry):
        o = g * _L
        acc = ub_w[pl.ds(o, _L)] + ib_w[pl.ds(o, _L)]
        for d in range(_D):
            acc += (ue_w[pl.ds(d * n_per_w + o, _L)]
                    * ie_w[pl.ds(d * n_per_w + o, _L)])
        pred = 4.0 / (1.0 + jnp.exp(-acc)) + 1.0
        out_v[pl.ds(o, _L)] = pred
        return carry

    lax.fori_loop(0, n_per_w // _L, group, 0)

    pltpu.sync_copy(out_v, out_hbm.at[pl.ds(base, n_per_w)])


def kernel(user_indices, item_indices, user_emb, item_emb, user_bias, item_bias):
    mesh = plsc.VectorSubcoreMesh(core_axis_name="c", subcore_axis_name="s")
    nw = mesh.num_cores * mesh.num_subcores
    n_per_w = _BATCH // nw

    f = pl.kernel(
        functools.partial(_mf_body, n_per_w=n_per_w, num_cores=mesh.num_cores),
        out_type=jax.ShapeDtypeStruct((_BATCH,), jnp.float32),
        mesh=mesh,
        compiler_params=pltpu.CompilerParams(needs_layout_passes=False),
        scratch_types=[
            pltpu.VMEM((n_per_w,), jnp.int32),
            pltpu.VMEM((n_per_w,), jnp.int32),
            pltpu.VMEM((n_per_w * _D,), jnp.float32),
            pltpu.VMEM((n_per_w * _D,), jnp.float32),
            pltpu.VMEM((n_per_w,), jnp.float32),
            pltpu.VMEM((n_per_w,), jnp.float32),
            pltpu.VMEM((n_per_w,), jnp.float32),
            pltpu.SemaphoreType.DMA,
            pltpu.SemaphoreType.DMA,
            pltpu.SemaphoreType.DMA,
            pltpu.SemaphoreType.DMA,
        ],
    )
    ucols = [user_emb[:, d] for d in range(_D)]
    icols = [item_emb[:, d] for d in range(_D)]
    return f(user_indices.astype(jnp.int32), item_indices.astype(jnp.int32),
             *ucols, *icols,
             user_bias.reshape(-1), item_bias.reshape(-1))


# speedup vs baseline: 1.6243x; 1.6243x over previous
import functools

import jax
import jax.numpy as jnp
from jax import lax
from jax.experimental import pallas as pl
from jax.experimental.pallas import tpu as pltpu
from jax.experimental.pallas import tpu_sc as plsc

_BATCH = 16384
_D = 32
_L = 16


def _mf_body(uidx_hbm, iidx_hbm, ue_hbm, ie_hbm, ub_hbm, ib_hbm, out_hbm,
             uidx_v, iidx_v, ue_v, ie_v, ub_v, ib_v, out_v,
             sem_ue, sem_ie, sem_ub, sem_ib, *, n_per_w, num_cores):
    wid = lax.axis_index("s") * num_cores + lax.axis_index("c")
    base = wid * n_per_w

    pltpu.sync_copy(uidx_hbm.at[pl.ds(base, n_per_w)], uidx_v)
    pltpu.sync_copy(iidx_hbm.at[pl.ds(base, n_per_w)], iidx_v)

    cu = pltpu.async_copy(ue_hbm.at[uidx_v], ue_v, sem_ue)
    ci = pltpu.async_copy(ie_hbm.at[iidx_v], ie_v, sem_ie)
    cub = pltpu.async_copy(ub_hbm.at[uidx_v], ub_v, sem_ub)
    cib = pltpu.async_copy(ib_hbm.at[iidx_v], ib_v, sem_ib)
    cu.wait()
    ci.wait()
    cub.wait()
    cib.wait()

    def group(g, carry):
        rows = lax.iota(jnp.int32, _L) + g * _L
        acc = (plsc.load_gather(ub_v, [rows])
               + plsc.load_gather(ib_v, [rows]))
        for d in range(_D):
            col = jnp.full((_L,), d, jnp.int32)
            acc += (plsc.load_gather(ue_v, [rows, col])
                    * plsc.load_gather(ie_v, [rows, col]))
        pred = 4.0 / (1.0 + jnp.exp(-acc)) + 1.0
        plsc.store_scatter(out_v, [rows], pred)
        return carry

    lax.fori_loop(0, n_per_w // _L, group, 0)

    pltpu.sync_copy(out_v, out_hbm.at[pl.ds(base, n_per_w)])


def kernel(user_indices, item_indices, user_emb, item_emb, user_bias, item_bias):
    mesh = plsc.VectorSubcoreMesh(core_axis_name="c", subcore_axis_name="s")
    nw = mesh.num_cores * mesh.num_subcores
    n_per_w = _BATCH // nw

    f = pl.kernel(
        functools.partial(_mf_body, n_per_w=n_per_w, num_cores=mesh.num_cores),
        out_type=jax.ShapeDtypeStruct((_BATCH,), jnp.float32),
        mesh=mesh,
        compiler_params=pltpu.CompilerParams(
            needs_layout_passes=False, use_tc_tiling_on_sc=False),
        scratch_types=[
            pltpu.VMEM((n_per_w,), jnp.int32),
            pltpu.VMEM((n_per_w,), jnp.int32),
            pltpu.VMEM((n_per_w, _D), jnp.float32),
            pltpu.VMEM((n_per_w, _D), jnp.float32),
            pltpu.VMEM((n_per_w,), jnp.float32),
            pltpu.VMEM((n_per_w,), jnp.float32),
            pltpu.VMEM((n_per_w,), jnp.float32),
            pltpu.SemaphoreType.DMA,
            pltpu.SemaphoreType.DMA,
            pltpu.SemaphoreType.DMA,
            pltpu.SemaphoreType.DMA,
        ],
    )
    return f(user_indices.astype(jnp.int32), item_indices.astype(jnp.int32),
             user_emb, item_emb,
             user_bias.reshape(-1), item_bias.reshape(-1))


# speedup vs baseline: 2.0306x; 1.2501x over previous
import functools

import jax
import jax.numpy as jnp
from jax import lax
from jax.experimental import pallas as pl
from jax.experimental.pallas import tpu as pltpu
from jax.experimental.pallas import tpu_sc as plsc

_BATCH = 16384
_D = 32
_L = 16
_ROWS_PER_BLOCK = 4
_BLOCK = _ROWS_PER_BLOCK * _D
_V = 1000000
_BN = 3968
_P = _BN // _ROWS_PER_BLOCK
_GRID = -(-_V // _BN)


def _repack_body(x_ref, o_ref):
    t = x_ref[...].T
    o_ref[...] = jnp.concatenate(
        [t[q * _P:(q + 1) * _P] for q in range(_ROWS_PER_BLOCK)], axis=1)


def _repack(xt):
    return pl.pallas_call(
        _repack_body,
        out_shape=jax.ShapeDtypeStruct((_GRID * _P, _BLOCK), jnp.float32),
        grid=(_GRID,),
        in_specs=[pl.BlockSpec((_D, _BN), lambda i: (0, i))],
        out_specs=pl.BlockSpec((_P, _BLOCK), lambda i: (i, 0)),
    )(xt)


def _mf_body(uidx_hbm, iidx_hbm, rm_u, rm_i, ub_flat, ib_flat, out_hbm,
             uidx_v, iidx_v, urid, irid, urows, irows, ub_w, ib_w, out_v,
             sem_ue, sem_ie, sem_ub, sem_ib, *, n_per_w, num_cores):
    wid = lax.axis_index("s") * num_cores + lax.axis_index("c")
    base = wid * n_per_w
    half = n_per_w // 2

    pltpu.sync_copy(uidx_hbm.at[pl.ds(base, n_per_w)], uidx_v)
    pltpu.sync_copy(iidx_hbm.at[pl.ds(base, n_per_w)], iidx_v)

    cub = pltpu.async_copy(ub_flat.at[uidx_v], ub_w, sem_ub)
    cib = pltpu.async_copy(ib_flat.at[iidx_v], ib_w, sem_ib)

    for h in range(2):
        off = h * half

        def build(g, carry):
            rows = lax.iota(jnp.int32, _L) + g * _L
            vu = plsc.load_gather(uidx_v, [rows + off])
            vi = plsc.load_gather(iidx_v, [rows + off])
            urid[pl.ds(g * _L, _L)] = (vu // _BN) * _P + (vu % _BN) % _P
            irid[pl.ds(g * _L, _L)] = (vi // _BN) * _P + (vi % _BN) % _P
            return carry

        lax.fori_loop(0, half // _L, build, 0)

        cu = pltpu.async_copy(rm_u.at[urid], urows, sem_ue)
        ci = pltpu.async_copy(rm_i.at[irid], irows, sem_ie)
        if h == 0:
            cub.wait()
            cib.wait()
        cu.wait()
        ci.wait()

        def group(g, carry):
            rows = lax.iota(jnp.int32, _L) + g * _L
            vu = plsc.load_gather(uidx_v, [rows + off])
            vi = plsc.load_gather(iidx_v, [rows + off])
            pu = ((vu % _BN) // _P) * _D
            pi = ((vi % _BN) // _P) * _D
            acc = (plsc.load_gather(ub_w, [rows + off])
                   + plsc.load_gather(ib_w, [rows + off]))
            for d in range(_D):
                acc += (plsc.load_gather(urows, [rows, pu + d])
                        * plsc.load_gather(irows, [rows, pi + d]))
            pred = 4.0 / (1.0 + jnp.exp(-acc)) + 1.0
            out_v[pl.ds(off + g * _L, _L)] = pred
            return carry

        lax.fori_loop(0, half // _L, group, 0)

    pltpu.sync_copy(out_v, out_hbm.at[pl.ds(base, n_per_w)])


def kernel(user_indices, item_indices, user_emb, item_emb, user_bias, item_bias):
    mesh = plsc.VectorSubcoreMesh(core_axis_name="c", subcore_axis_name="s")
    nw = mesh.num_cores * mesh.num_subcores
    n_per_w = _BATCH // nw
    half = n_per_w // 2

    f = pl.kernel(
        functools.partial(_mf_body, n_per_w=n_per_w, num_cores=mesh.num_cores),
        out_type=jax.ShapeDtypeStruct((_BATCH,), jnp.float32),
        mesh=mesh,
        compiler_params=pltpu.CompilerParams(
            needs_layout_passes=False, use_tc_tiling_on_sc=True),
        scratch_types=[
            pltpu.VMEM((n_per_w,), jnp.int32),
            pltpu.VMEM((n_per_w,), jnp.int32),
            pltpu.VMEM((half,), jnp.int32),
            pltpu.VMEM((half,), jnp.int32),
            pltpu.VMEM((half, _BLOCK), jnp.float32),
            pltpu.VMEM((half, _BLOCK), jnp.float32),
            pltpu.VMEM((n_per_w,), jnp.float32),
            pltpu.VMEM((n_per_w,), jnp.float32),
            pltpu.VMEM((n_per_w,), jnp.float32),
            pltpu.SemaphoreType.DMA,
            pltpu.SemaphoreType.DMA,
            pltpu.SemaphoreType.DMA,
            pltpu.SemaphoreType.DMA,
        ],
    )
    rm_u = _repack(user_emb.T)
    rm_i = _repack(item_emb.T)
    return f(user_indices.astype(jnp.int32), item_indices.astype(jnp.int32),
             rm_u, rm_i, user_bias.reshape(-1), item_bias.reshape(-1))


# speedup vs baseline: 2.3275x; 1.1462x over previous
import functools

import jax
import jax.numpy as jnp
from jax import lax
from jax.experimental import pallas as pl
from jax.experimental.pallas import tpu as pltpu
from jax.experimental.pallas import tpu_sc as plsc

_BATCH = 16384
_D = 32
_L = 16
_ROWS_PER_BLOCK = 4
_BLOCK = _ROWS_PER_BLOCK * _D
_V = 1000000
_BN = 7936
_P = _BN // _ROWS_PER_BLOCK
_GRID = -(-_V // _BN)


def _repack_body(x_ref, o_ref):
    t = x_ref[...].T
    o_ref[...] = jnp.concatenate(
        [t[q * _P:(q + 1) * _P] for q in range(_ROWS_PER_BLOCK)], axis=1)


def _repack(xt):
    return pl.pallas_call(
        _repack_body,
        out_shape=jax.ShapeDtypeStruct((_GRID * _P, _BLOCK), jnp.float32),
        grid=(_GRID,),
        in_specs=[pl.BlockSpec((_D, _BN), lambda i: (0, i))],
        out_specs=pl.BlockSpec((_P, _BLOCK), lambda i: (i, 0)),
    )(xt)


def _mf_body(uidx_hbm, iidx_hbm, rm_u, rm_i, ub_flat, ib_flat, out_hbm,
             uidx_v, iidx_v, urid, irid, urows, irows, ub_w, ib_w, out_v,
             sem_ue, sem_ie, sem_ub, sem_ib, *, n_per_w, num_cores):
    wid = lax.axis_index("s") * num_cores + lax.axis_index("c")
    base = wid * n_per_w
    half = n_per_w // 2

    pltpu.sync_copy(uidx_hbm.at[pl.ds(base, n_per_w)], uidx_v)
    pltpu.sync_copy(iidx_hbm.at[pl.ds(base, n_per_w)], iidx_v)

    cub = pltpu.async_copy(ub_flat.at[uidx_v], ub_w, sem_ub)
    cib = pltpu.async_copy(ib_flat.at[iidx_v], ib_w, sem_ib)

    for h in range(2):
        off = h * half

        def build(g, carry):
            rows = lax.iota(jnp.int32, _L) + g * _L
            vu = plsc.load_gather(uidx_v, [rows + off])
            vi = plsc.load_gather(iidx_v, [rows + off])
            urid[pl.ds(g * _L, _L)] = (vu // _BN) * _P + (vu % _BN) % _P
            irid[pl.ds(g * _L, _L)] = (vi // _BN) * _P + (vi % _BN) % _P
            return carry

        lax.fori_loop(0, half // _L, build, 0)

        cu = pltpu.async_copy(rm_u.at[urid], urows, sem_ue)
        ci = pltpu.async_copy(rm_i.at[irid], irows, sem_ie)
        if h == 0:
            cub.wait()
            cib.wait()
        cu.wait()
        ci.wait()

        def group(g, carry):
            rows = lax.iota(jnp.int32, _L) + g * _L
            vu = plsc.load_gather(uidx_v, [rows + off])
            vi = plsc.load_gather(iidx_v, [rows + off])
            pu = ((vu % _BN) // _P) * _D
            pi = ((vi % _BN) // _P) * _D
            acc = (plsc.load_gather(ub_w, [rows + off])
                   + plsc.load_gather(ib_w, [rows + off]))
            for d in range(_D):
                acc += (plsc.load_gather(urows, [rows, pu + d])
                        * plsc.load_gather(irows, [rows, pi + d]))
            pred = 4.0 / (1.0 + jnp.exp(-acc)) + 1.0
            out_v[pl.ds(off + g * _L, _L)] = pred
            return carry

        lax.fori_loop(0, half // _L, group, 0)

    pltpu.sync_copy(out_v, out_hbm.at[pl.ds(base, n_per_w)])


def kernel(user_indices, item_indices, user_emb, item_emb, user_bias, item_bias):
    mesh = plsc.VectorSubcoreMesh(core_axis_name="c", subcore_axis_name="s")
    nw = mesh.num_cores * mesh.num_subcores
    n_per_w = _BATCH // nw
    half = n_per_w // 2

    f = pl.kernel(
        functools.partial(_mf_body, n_per_w=n_per_w, num_cores=mesh.num_cores),
        out_type=jax.ShapeDtypeStruct((_BATCH,), jnp.float32),
        mesh=mesh,
        compiler_params=pltpu.CompilerParams(
            needs_layout_passes=False, use_tc_tiling_on_sc=True),
        scratch_types=[
            pltpu.VMEM((n_per_w,), jnp.int32),
            pltpu.VMEM((n_per_w,), jnp.int32),
            pltpu.VMEM((half,), jnp.int32),
            pltpu.VMEM((half,), jnp.int32),
            pltpu.VMEM((half, _BLOCK), jnp.float32),
            pltpu.VMEM((half, _BLOCK), jnp.float32),
            pltpu.VMEM((n_per_w,), jnp.float32),
            pltpu.VMEM((n_per_w,), jnp.float32),
            pltpu.VMEM((n_per_w,), jnp.float32),
            pltpu.SemaphoreType.DMA,
            pltpu.SemaphoreType.DMA,
            pltpu.SemaphoreType.DMA,
            pltpu.SemaphoreType.DMA,
        ],
    )
    rm_u = _repack(user_emb.T)
    rm_i = _repack(item_emb.T)
    return f(user_indices.astype(jnp.int32), item_indices.astype(jnp.int32),
             rm_u, rm_i, user_bias.reshape(-1), item_bias.reshape(-1))


# speedup vs baseline: 2.3803x; 1.0227x over previous
import functools

import jax
import jax.numpy as jnp
from jax import lax
from jax.experimental import pallas as pl
from jax.experimental.pallas import tpu as pltpu
from jax.experimental.pallas import tpu_sc as plsc

_BATCH = 16384
_D = 32
_L = 16
_ROWS_PER_BLOCK = 4
_BLOCK = _ROWS_PER_BLOCK * _D
_V = 1000000
_BN = 31744
_P = _BN // _ROWS_PER_BLOCK
_GRID = -(-_V // _BN)


def _repack_body(x_ref, o_ref):
    t = x_ref[...].T
    o_ref[...] = jnp.concatenate(
        [t[q * _P:(q + 1) * _P] for q in range(_ROWS_PER_BLOCK)], axis=1)


def _repack(xt):
    return pl.pallas_call(
        _repack_body,
        out_shape=jax.ShapeDtypeStruct((_GRID * _P, _BLOCK), jnp.float32),
        grid=(_GRID,),
        in_specs=[pl.BlockSpec((_D, _BN), lambda i: (0, i))],
        out_specs=pl.BlockSpec((_P, _BLOCK), lambda i: (i, 0)),
    )(xt)


def _mf_body(uidx_hbm, iidx_hbm, rm_u, rm_i, ub_flat, ib_flat, out_hbm,
             uidx_v, iidx_v, urid, irid, urows, irows, ub_w, ib_w, out_v,
             sem_ue, sem_ie, sem_ub, sem_ib, *, n_per_w, num_cores):
    wid = lax.axis_index("s") * num_cores + lax.axis_index("c")
    base = wid * n_per_w
    half = n_per_w // 2

    pltpu.sync_copy(uidx_hbm.at[pl.ds(base, n_per_w)], uidx_v)
    pltpu.sync_copy(iidx_hbm.at[pl.ds(base, n_per_w)], iidx_v)

    cub = pltpu.async_copy(ub_flat.at[uidx_v], ub_w, sem_ub)
    cib = pltpu.async_copy(ib_flat.at[iidx_v], ib_w, sem_ib)

    for h in range(2):
        off = h * half

        def build(g, carry):
            rows = lax.iota(jnp.int32, _L) + g * _L
            vu = plsc.load_gather(uidx_v, [rows + off])
            vi = plsc.load_gather(iidx_v, [rows + off])
            urid[pl.ds(g * _L, _L)] = (vu // _BN) * _P + (vu % _BN) % _P
            irid[pl.ds(g * _L, _L)] = (vi // _BN) * _P + (vi % _BN) % _P
            return carry

        lax.fori_loop(0, half // _L, build, 0)

        cu = pltpu.async_copy(rm_u.at[urid], urows, sem_ue)
        ci = pltpu.async_copy(rm_i.at[irid], irows, sem_ie)
        if h == 0:
            cub.wait()
            cib.wait()
        cu.wait()
        ci.wait()

        def group(g, carry):
            rows = lax.iota(jnp.int32, _L) + g * _L
            vu = plsc.load_gather(uidx_v, [rows + off])
            vi = plsc.load_gather(iidx_v, [rows + off])
            pu = ((vu % _BN) // _P) * _D
            pi = ((vi % _BN) // _P) * _D
            acc = (plsc.load_gather(ub_w, [rows + off])
                   + plsc.load_gather(ib_w, [rows + off]))
            for d in range(_D):
                acc += (plsc.load_gather(urows, [rows, pu + d])
                        * plsc.load_gather(irows, [rows, pi + d]))
            pred = 4.0 / (1.0 + jnp.exp(-acc)) + 1.0
            out_v[pl.ds(off + g * _L, _L)] = pred
            return carry

        lax.fori_loop(0, half // _L, group, 0)

    pltpu.sync_copy(out_v, out_hbm.at[pl.ds(base, n_per_w)])


def kernel(user_indices, item_indices, user_emb, item_emb, user_bias, item_bias):
    mesh = plsc.VectorSubcoreMesh(core_axis_name="c", subcore_axis_name="s")
    nw = mesh.num_cores * mesh.num_subcores
    n_per_w = _BATCH // nw
    half = n_per_w // 2

    f = pl.kernel(
        functools.partial(_mf_body, n_per_w=n_per_w, num_cores=mesh.num_cores),
        out_type=jax.ShapeDtypeStruct((_BATCH,), jnp.float32),
        mesh=mesh,
        compiler_params=pltpu.CompilerParams(
            needs_layout_passes=False, use_tc_tiling_on_sc=True),
        scratch_types=[
            pltpu.VMEM((n_per_w,), jnp.int32),
            pltpu.VMEM((n_per_w,), jnp.int32),
            pltpu.VMEM((half,), jnp.int32),
            pltpu.VMEM((half,), jnp.int32),
            pltpu.VMEM((half, _BLOCK), jnp.float32),
            pltpu.VMEM((half, _BLOCK), jnp.float32),
            pltpu.VMEM((n_per_w,), jnp.float32),
            pltpu.VMEM((n_per_w,), jnp.float32),
            pltpu.VMEM((n_per_w,), jnp.float32),
            pltpu.SemaphoreType.DMA,
            pltpu.SemaphoreType.DMA,
            pltpu.SemaphoreType.DMA,
            pltpu.SemaphoreType.DMA,
        ],
    )
    rm_u = _repack(user_emb.T)
    rm_i = _repack(item_emb.T)
    return f(user_indices.astype(jnp.int32), item_indices.astype(jnp.int32),
             rm_u, rm_i, user_bias.reshape(-1), item_bias.reshape(-1))


# speedup vs baseline: 3.7290x; 1.5666x over previous
import functools

import jax
import jax.numpy as jnp
from jax import lax
from jax.experimental import pallas as pl
from jax.experimental.pallas import tpu as pltpu
from jax.experimental.pallas import tpu_sc as plsc

_BATCH = 16384
_D = 32
_L = 16
_ROWS_PER_BLOCK = 4
_BLOCK = _ROWS_PER_BLOCK * _D
_V = 1000000
_BN = 31744
_P = _BN // _ROWS_PER_BLOCK
_GRID = -(-_V // _BN)


def _repack_body(x_ref, o_ref):
    t = x_ref[...].astype(jnp.bfloat16).T.astype(jnp.float32)
    o_ref[...] = jnp.concatenate(
        [t[q * _P:(q + 1) * _P] for q in range(_ROWS_PER_BLOCK)], axis=1)


def _repack(xt):
    return pl.pallas_call(
        _repack_body,
        out_shape=jax.ShapeDtypeStruct((_GRID * _P, _BLOCK), jnp.float32),
        grid=(_GRID,),
        in_specs=[pl.BlockSpec((_D, _BN), lambda i: (0, i))],
        out_specs=pl.BlockSpec((_P, _BLOCK), lambda i: (i, 0)),
    )(xt)


def _mf_body(uidx_hbm, iidx_hbm, rm_u, rm_i, ub_flat, ib_flat, out_hbm,
             uidx_v, iidx_v, urid, irid, urows, irows, ub_w, ib_w, out_v,
             sem_ue, sem_ie, sem_ub, sem_ib, *, n_per_w, num_cores):
    wid = lax.axis_index("s") * num_cores + lax.axis_index("c")
    base = wid * n_per_w
    half = n_per_w // 2

    pltpu.sync_copy(uidx_hbm.at[pl.ds(base, n_per_w)], uidx_v)
    pltpu.sync_copy(iidx_hbm.at[pl.ds(base, n_per_w)], iidx_v)

    cub = pltpu.async_copy(ub_flat.at[uidx_v], ub_w, sem_ub)
    cib = pltpu.async_copy(ib_flat.at[iidx_v], ib_w, sem_ib)

    for h in range(2):
        off = h * half

        def build(g, carry):
            rows = lax.iota(jnp.int32, _L) + g * _L
            vu = plsc.load_gather(uidx_v, [rows + off])
            vi = plsc.load_gather(iidx_v, [rows + off])
            urid[pl.ds(g * _L, _L)] = (vu // _BN) * _P + (vu % _BN) % _P
            irid[pl.ds(g * _L, _L)] = (vi // _BN) * _P + (vi % _BN) % _P
            return carry

        lax.fori_loop(0, half // _L, build, 0)

        cu = pltpu.async_copy(rm_u.at[urid], urows, sem_ue)
        ci = pltpu.async_copy(rm_i.at[irid], irows, sem_ie)
        if h == 0:
            cub.wait()
            cib.wait()
        cu.wait()
        ci.wait()

        def group(g, carry):
            rows = lax.iota(jnp.int32, _L) + g * _L
            vu = plsc.load_gather(uidx_v, [rows + off])
            vi = plsc.load_gather(iidx_v, [rows + off])
            pu = ((vu % _BN) // _P) * _D
            pi = ((vi % _BN) // _P) * _D
            acc = (plsc.load_gather(ub_w, [rows + off])
                   + plsc.load_gather(ib_w, [rows + off]))
            for d in range(_D):
                acc += (plsc.load_gather(urows, [rows, pu + d])
                        * plsc.load_gather(irows, [rows, pi + d]))
            pred = 4.0 / (1.0 + jnp.exp(-acc)) + 1.0
            out_v[pl.ds(off + g * _L, _L)] = pred
            return carry

        lax.fori_loop(0, half // _L, group, 0)

    pltpu.sync_copy(out_v, out_hbm.at[pl.ds(base, n_per_w)])


def kernel(user_indices, item_indices, user_emb, item_emb, user_bias, item_bias):
    mesh = plsc.VectorSubcoreMesh(core_axis_name="c", subcore_axis_name="s")
    nw = mesh.num_cores * mesh.num_subcores
    n_per_w = _BATCH // nw
    half = n_per_w // 2

    f = pl.kernel(
        functools.partial(_mf_body, n_per_w=n_per_w, num_cores=mesh.num_cores),
        out_type=jax.ShapeDtypeStruct((_BATCH,), jnp.float32),
        mesh=mesh,
        compiler_params=pltpu.CompilerParams(
            needs_layout_passes=False, use_tc_tiling_on_sc=True),
        scratch_types=[
            pltpu.VMEM((n_per_w,), jnp.int32),
            pltpu.VMEM((n_per_w,), jnp.int32),
            pltpu.VMEM((half,), jnp.int32),
            pltpu.VMEM((half,), jnp.int32),
            pltpu.VMEM((half, _BLOCK), jnp.float32),
            pltpu.VMEM((half, _BLOCK), jnp.float32),
            pltpu.VMEM((n_per_w,), jnp.float32),
            pltpu.VMEM((n_per_w,), jnp.float32),
            pltpu.VMEM((n_per_w,), jnp.float32),
            pltpu.SemaphoreType.DMA,
            pltpu.SemaphoreType.DMA,
            pltpu.SemaphoreType.DMA,
            pltpu.SemaphoreType.DMA,
        ],
    )
    rm_u = _repack(user_emb.T)
    rm_i = _repack(item_emb.T)
    return f(user_indices.astype(jnp.int32), item_indices.astype(jnp.int32),
             rm_u, rm_i, user_bias.reshape(-1), item_bias.reshape(-1))
